# baseline (speedup 1.0000x reference)
import functools

import jax
import jax.numpy as jnp
from jax import lax
from jax.experimental import pallas as pl
from jax.experimental.pallas import tpu as pltpu
from jax.experimental.pallas import tpu_sc as plsc

_NUM_CORES = 2
_NUM_SUBCORES = 16
_NUM_WORKERS = _NUM_CORES * _NUM_SUBCORES
_CHUNK = 128


@functools.lru_cache(maxsize=None)
def _build(B, V, D):
    assert B % _NUM_WORKERS == 0
    b_per_w = B // _NUM_WORKERS
    n_chunks = -(-b_per_w // _CHUNK)
    assert b_per_w % _CHUNK == 0
    mesh = plsc.VectorSubcoreMesh(core_axis_name="c", subcore_axis_name="s")

    @functools.partial(
        pl.kernel,
        mesh=mesh,
        out_type=jax.ShapeDtypeStruct((B, D), jnp.float32),
        scratch_types=[
            pltpu.VMEM((b_per_w,), jnp.int32),
            pltpu.VMEM((b_per_w, D), jnp.float32),
            pltpu.SemaphoreType.DMA,
        ],
        compiler_params=pltpu.CompilerParams(use_tc_tiling_on_sc=False),
    )
    def gather_kernel(msg_hbm, table_hbm, out_hbm, idx_v, rows_v, sem):
        wid = lax.axis_index("s") * _NUM_CORES + lax.axis_index("c")
        base = wid * b_per_w
        pltpu.sync_copy(msg_hbm.at[pl.ds(base, b_per_w)], idx_v)
        copies = [
            pltpu.async_copy(
                table_hbm.at[idx_v.at[pl.ds(c * _CHUNK, _CHUNK)]],
                rows_v.at[pl.ds(c * _CHUNK, _CHUNK)],
                sem,
            )
            for c in range(n_chunks)
        ]
        for cp in copies:
            cp.wait()
        pltpu.sync_copy(rows_v, out_hbm.at[pl.ds(base, b_per_w)])

    return gather_kernel


def kernel(message, embedding_table):
    B, = message.shape
    V, D = embedding_table.shape
    return _build(B, V, D)(message.astype(jnp.int32), embedding_table)


# speedup vs baseline: 1.0232x; 1.0232x over previous
import functools

import jax
import jax.numpy as jnp
from jax import lax
from jax.experimental import pallas as pl
from jax.experimental.pallas import tpu as pltpu
from jax.experimental.pallas import tpu_sc as plsc

_NUM_CORES = 2
_NUM_SUBCORES = 16
_NUM_WORKERS = _NUM_CORES * _NUM_SUBCORES
_UNROLL = 16


@functools.lru_cache(maxsize=None)
def _build(B, V, D):
    assert B % (_NUM_WORKERS * _UNROLL) == 0
    b_per_w = B // _NUM_WORKERS
    n_steps = b_per_w // _UNROLL
    mesh = plsc.VectorSubcoreMesh(core_axis_name="c", subcore_axis_name="s")

    @functools.partial(
        pl.kernel,
        mesh=mesh,
        out_type=jax.ShapeDtypeStruct((B, D), jnp.float32),
        scratch_types=[
            pltpu.VMEM((b_per_w,), jnp.int32),
            pltpu.SemaphoreType.DMA,
        ],
    )
    def gather_kernel(msg_hbm, table_hbm, out_hbm, idx_v, sem):
        wid = lax.axis_index("s") * _NUM_CORES + lax.axis_index("c")
        base = wid * b_per_w
        pltpu.sync_copy(msg_hbm.at[pl.ds(base, b_per_w)], idx_v)

        def step(j, carry):
            vec = idx_v[pl.ds(j * _UNROLL, _UNROLL)]
            for u in range(_UNROLL):
                row = vec[u]
                pltpu.async_copy(
                    table_hbm.at[pl.ds(row, 1), :],
                    out_hbm.at[pl.ds(base + j * _UNROLL + u, 1), :],
                    sem,
                )
            return carry

        lax.fori_loop(0, n_steps, step, 0)
        pltpu.make_async_copy(
            table_hbm.at[pl.ds(0, b_per_w), :],
            out_hbm.at[pl.ds(base, b_per_w), :],
            sem,
        ).wait()

    return gather_kernel


def kernel(message, embedding_table):
    B, = message.shape
    V, D = embedding_table.shape
    return _build(B, V, D)(message.astype(jnp.int32), embedding_table)


# speedup vs baseline: 1.7068x; 1.6681x over previous
import functools

import jax
import jax.numpy as jnp
from jax import lax
from jax.experimental import pallas as pl
from jax.experimental.pallas import tpu as pltpu
from jax.experimental.pallas import tpu_sc as plsc

_NUM_CORES = 2
_NUM_SUBCORES = 16
_NUM_WORKERS = _NUM_CORES * _NUM_SUBCORES
_LANES = 16


@functools.lru_cache(maxsize=None)
def _build(B, V, D):
    assert B % (_NUM_WORKERS * _LANES) == 0
    b_per_w = B // _NUM_WORKERS
    n_chunks = b_per_w // _LANES
    mesh = plsc.VectorSubcoreMesh(core_axis_name="c", subcore_axis_name="s")

    @functools.partial(
        pl.kernel,
        mesh=mesh,
        out_type=jax.ShapeDtypeStruct((B, D), jnp.float32),
        scratch_types=[
            pltpu.VMEM((b_per_w,), jnp.int32),
            pltpu.VMEM((b_per_w, D), jnp.float32),
            pltpu.SemaphoreType.DMA,
        ],
    )
    def gather_kernel(msg_hbm, tbl_hbm, out_hbm, idx_v, rows_v, sem):
        wid = lax.axis_index("s") * _NUM_CORES + lax.axis_index("c")
        base = wid * b_per_w
        pltpu.sync_copy(msg_hbm.at[pl.ds(base, b_per_w)], idx_v)

        def fire_chunk(c, carry):
            idxvec = idx_v[pl.ds(c * _LANES, _LANES)]
            for u in range(_LANES):
                row = idxvec[u]
                pltpu.async_copy(
                    tbl_hbm.at[pl.ds(row, 1), :],
                    rows_v.at[pl.ds(c * _LANES + u, 1), :],
                    sem,
                )
            return carry

        lax.fori_loop(0, n_chunks, fire_chunk, 0)

        def drain(k, carry):
            pltpu.make_async_copy(
                tbl_hbm.at[pl.ds(0, 1), :],
                rows_v.at[pl.ds(0, 1), :],
                sem,
            ).wait()
            return carry

        lax.fori_loop(0, b_per_w, drain, 0)
        pltpu.sync_copy(rows_v, out_hbm.at[pl.ds(base, b_per_w)])

    return gather_kernel


def kernel(message, embedding_table):
    B, = message.shape
    V, D = embedding_table.shape
    return _build(B, V, D)(message.astype(jnp.int32), embedding_table)


# speedup vs baseline: 1.7074x; 1.0004x over previous
import functools

import jax
import jax.numpy as jnp
from jax import lax
from jax.experimental import pallas as pl
from jax.experimental.pallas import tpu as pltpu
from jax.experimental.pallas import tpu_sc as plsc

_NUM_CORES = 2
_NUM_SUBCORES = 16
_NUM_WORKERS = _NUM_CORES * _NUM_SUBCORES
_LANES = 16


@functools.lru_cache(maxsize=None)
def _build(B, V, D):
    assert B % (_NUM_WORKERS * _LANES) == 0
    b_per_w = B // _NUM_WORKERS
    n_chunks = b_per_w // _LANES
    mesh = plsc.VectorSubcoreMesh(core_axis_name="c", subcore_axis_name="s")

    @functools.partial(
        pl.kernel,
        mesh=mesh,
        out_type=jax.ShapeDtypeStruct((B, D), jnp.float32),
        scratch_types=[
            pltpu.VMEM((b_per_w,), jnp.int32),
            pltpu.VMEM((b_per_w, D), jnp.float32),
            pltpu.SemaphoreType.DMA,
        ],
        compiler_params=pltpu.CompilerParams(needs_layout_passes=False),
    )
    def gather_kernel(msg_hbm, tbl_hbm, out_hbm, idx_v, rows_v, sem):
        wid = lax.axis_index("s") * _NUM_CORES + lax.axis_index("c")
        base = wid * b_per_w
        pltpu.sync_copy(msg_hbm.at[pl.ds(base, b_per_w)], idx_v)

        def fire_chunk(c, carry):
            idxvec = idx_v[pl.ds(c * _LANES, _LANES)]
            for u in range(_LANES):
                row = idxvec[u]
                pltpu.async_copy(
                    tbl_hbm.at[pl.ds(row, 1), :],
                    rows_v.at[pl.ds(c * _LANES + u, 1), :],
                    sem,
                )
            return carry

        lax.fori_loop(0, n_chunks, fire_chunk, 0)

        def drain(k, carry):
            pltpu.make_async_copy(
                tbl_hbm.at[pl.ds(0, 1), :],
                rows_v.at[pl.ds(0, 1), :],
                sem,
            ).wait()
            return carry

        lax.fori_loop(0, b_per_w, drain, 0)
        pltpu.sync_copy(rows_v, out_hbm.at[pl.ds(base, b_per_w)])

    return gather_kernel


def kernel(message, embedding_table):
    B, = message.shape
    V, D = embedding_table.shape
    return _build(B, V, D)(message.astype(jnp.int32), embedding_table)


# speedup vs baseline: 2.5554x; 1.4966x over previous
import functools

import jax
import jax.numpy as jnp
from jax import lax
from jax.experimental import pallas as pl
from jax.experimental.pallas import tpu as pltpu
from jax.experimental.pallas import tpu_sc as plsc

_NUM_CORES = 2
_NUM_SUBCORES = 16
_NUM_WORKERS = _NUM_CORES * _NUM_SUBCORES
_LANES = 16


@functools.lru_cache(maxsize=None)
def _build(B, V, D):
    assert B % (_NUM_WORKERS * _LANES) == 0 and V % 8 == 0
    b_per_w = B // _NUM_WORKERS
    n_chunks = b_per_w // _LANES
    mesh = plsc.VectorSubcoreMesh(core_axis_name="c", subcore_axis_name="s")

    @functools.partial(
        pl.kernel,
        mesh=mesh,
        out_type=jax.ShapeDtypeStruct((B, D), jnp.float32),
        scratch_types=[
            pltpu.VMEM((b_per_w,), jnp.int32),
            pltpu.VMEM((b_per_w, D), jnp.float32),
            pltpu.SemaphoreType.DMA,
        ],
    )
    def gather_kernel(msg_hbm, tbl_hbm, out_hbm, idx_v, rows_v, sem):
        wid = lax.axis_index("s") * _NUM_CORES + lax.axis_index("c")
        base = wid * b_per_w
        pltpu.sync_copy(msg_hbm.at[pl.ds(base, b_per_w)], idx_v)

        def fire_chunk(c, carry):
            idxvec = idx_v[pl.ds(c * _LANES, _LANES)]
            blkvec = lax.shift_right_logical(idxvec, 3)
            subvec = jnp.bitwise_and(idxvec, 7)
            for u in range(_LANES):
                pltpu.async_copy(
                    tbl_hbm.at[pl.ds(blkvec[u], 1), subvec[u], :],
                    rows_v.at[pl.ds(c * _LANES + u, 1), :],
                    sem,
                )
            return carry

        lax.fori_loop(0, n_chunks, fire_chunk, 0)

        def drain(k, carry):
            pltpu.make_async_copy(
                tbl_hbm.at[pl.ds(0, 1), 0, :],
                rows_v.at[pl.ds(0, 1), :],
                sem,
            ).wait()
            return carry

        lax.fori_loop(0, b_per_w, drain, 0)
        pltpu.sync_copy(rows_v, out_hbm.at[pl.ds(base, b_per_w)])

    return gather_kernel


def kernel(message, embedding_table):
    B, = message.shape
    V, D = embedding_table.shape
    tbl3 = embedding_table.reshape(V // 8, 8, D)
    return _build(B, V, D)(message.astype(jnp.int32), tbl3)
